# baseline (device time: 255155 ns/iter reference)
import jax
import jax.numpy as jnp
from jax import lax
from jax.experimental import pallas as pl
from jax.experimental.pallas import tpu as pltpu

N_DEV = 8
E_LOCAL = 8
N_TOK = 2048
D_MODEL = 1024
CHUNK = N_TOK // N_DEV


def _moe_partial(x, expert_W, c):

    def body(x_ref, w_ref, c_ref, out_ref):
        e = pl.program_id(0)
        prod = jnp.dot(
            x_ref[:, :], w_ref[0, :, :], preferred_element_type=jnp.float32
        )
        contrib = prod * c_ref[e][:, None]

        @pl.when(e == 0)
        def _():
            out_ref[:, :] = contrib

        @pl.when(e != 0)
        def _():
            out_ref[:, :] = out_ref[:, :] + contrib

    return pl.pallas_call(
        body,
        grid=(E_LOCAL,),
        in_specs=[
            pl.BlockSpec((N_TOK, D_MODEL), lambda e: (0, 0)),
            pl.BlockSpec((1, D_MODEL, D_MODEL), lambda e: (e, 0, 0)),
            pl.BlockSpec((E_LOCAL, N_TOK), lambda e: (0, 0)),
        ],
        out_specs=pl.BlockSpec((N_TOK, D_MODEL), lambda e: (0, 0)),
        out_shape=jax.ShapeDtypeStruct((N_TOK, D_MODEL), jnp.float32),
    )(x, expert_W, c)


def _ring_all_reduce(partial):

    def body(p_ref, out_ref, rs_buf, rs_send, rs_recv, ag_send, ag_recv):
        my = lax.axis_index("i")
        left = (my - 1) % N_DEV
        right = (my + 1) % N_DEV

        barrier_sem = pltpu.get_barrier_semaphore()
        for nbr in (left, right):
            pl.semaphore_signal(
                barrier_sem,
                inc=1,
                device_id=(nbr,),
                device_id_type=pl.DeviceIdType.MESH,
            )
        pl.semaphore_wait(barrier_sem, 2)

        descs = []

        for s in range(N_DEV - 1):
            cs = (my - s) % N_DEV
            cr = (my - s - 1) % N_DEV
            if s == 0:
                src = p_ref.at[pl.ds(cs * CHUNK, CHUNK), :]
            else:
                src = rs_buf.at[s - 1]
            rdma = pltpu.make_async_remote_copy(
                src_ref=src,
                dst_ref=rs_buf.at[s],
                send_sem=rs_send.at[s],
                recv_sem=rs_recv.at[s],
                device_id=(right,),
                device_id_type=pl.DeviceIdType.MESH,
            )
            rdma.start()
            rdma.wait_recv()
            descs.append(rdma)
            rs_buf[s, :, :] = rs_buf[s, :, :] + p_ref[pl.ds(cr * CHUNK, CHUNK), :]

        own = (my + 1) % N_DEV
        out_ref[pl.ds(own * CHUNK, CHUNK), :] = rs_buf[N_DEV - 2, :, :]

        for h in range(N_DEV - 1):
            ca = (my + 1 - h) % N_DEV
            rdma = pltpu.make_async_remote_copy(
                src_ref=out_ref.at[pl.ds(ca * CHUNK, CHUNK), :],
                dst_ref=out_ref.at[pl.ds(ca * CHUNK, CHUNK), :],
                send_sem=ag_send.at[h],
                recv_sem=ag_recv.at[h],
                device_id=(right,),
                device_id_type=pl.DeviceIdType.MESH,
            )
            rdma.start()
            rdma.wait_recv()
            descs.append(rdma)

        for rdma in descs:
            rdma.wait_send()

    return pl.pallas_call(
        body,
        out_shape=jax.ShapeDtypeStruct((N_TOK, D_MODEL), jnp.float32),
        in_specs=[pl.BlockSpec(memory_space=pltpu.VMEM)],
        out_specs=pl.BlockSpec(memory_space=pltpu.VMEM),
        scratch_shapes=[
            pltpu.VMEM((N_DEV - 1, CHUNK, D_MODEL), jnp.float32),
            pltpu.SemaphoreType.DMA((N_DEV - 1,)),
            pltpu.SemaphoreType.DMA((N_DEV - 1,)),
            pltpu.SemaphoreType.DMA((N_DEV - 1,)),
            pltpu.SemaphoreType.DMA((N_DEV - 1,)),
        ],
        compiler_params=pltpu.CompilerParams(collective_id=0),
    )(partial)


def kernel(x, router_W, route_idx, expert_W):
    my = lax.axis_index("i")

    scores = x @ router_W
    m = scores.max(axis=-1, keepdims=True)
    p = jnp.exp(scores - m)
    p = p / p.sum(axis=-1, keepdims=True)
    g = jnp.take_along_axis(p, route_idx, axis=1)
    w = g / g.sum(axis=-1, keepdims=True)

    local_e = my * E_LOCAL + jnp.arange(E_LOCAL, dtype=jnp.int32)
    hit0 = route_idx[:, 0][None, :] == local_e[:, None]
    hit1 = route_idx[:, 1][None, :] == local_e[:, None]
    c = (w[:, 0][None, :] * hit0 + w[:, 1][None, :] * hit1).astype(jnp.float32)

    partial = _moe_partial(x, expert_W, c)
    return _ring_all_reduce(partial)


# device time: 182799 ns/iter; 1.3958x vs baseline; 1.3958x over previous
import jax
import jax.numpy as jnp
from jax import lax
from jax.experimental import pallas as pl
from jax.experimental.pallas import tpu as pltpu

N_DEV = 8
E_LOCAL = 8
N_TOK = 2048
D_MODEL = 1024
CHUNK = N_TOK // N_DEV


def _moe_partial(x, expert_W, c):

    def body(x_ref, w_ref, c_ref, out_ref):
        e = pl.program_id(0)
        prod = jnp.dot(
            x_ref[:, :], w_ref[0, :, :], preferred_element_type=jnp.float32
        )
        contrib = prod * c_ref[e][:, None]

        @pl.when(e == 0)
        def _():
            out_ref[:, :] = contrib

        @pl.when(e != 0)
        def _():
            out_ref[:, :] = out_ref[:, :] + contrib

    return pl.pallas_call(
        body,
        grid=(E_LOCAL,),
        in_specs=[
            pl.BlockSpec((N_TOK, D_MODEL), lambda e: (0, 0)),
            pl.BlockSpec((1, D_MODEL, D_MODEL), lambda e: (e, 0, 0)),
            pl.BlockSpec((E_LOCAL, N_TOK), lambda e: (0, 0)),
        ],
        out_specs=pl.BlockSpec((N_TOK, D_MODEL), lambda e: (0, 0)),
        out_shape=jax.ShapeDtypeStruct((N_TOK, D_MODEL), jnp.float32),
    )(x, expert_W, c)


def _ring_all_reduce(partial):
    HC = CHUNK // 2

    def body(p_ref, out_ref, cwb, ccwb, rs_send, rs_recv, ag_send, ag_recv):
        my = lax.axis_index("i")
        left = (my - 1) % N_DEV
        right = (my + 1) % N_DEV

        barrier_sem = pltpu.get_barrier_semaphore()
        for nbr in (left, right):
            pl.semaphore_signal(
                barrier_sem,
                inc=1,
                device_id=(nbr,),
                device_id_type=pl.DeviceIdType.MESH,
            )
        pl.semaphore_wait(barrier_sem, 2)

        descs = []

        for s in range(N_DEV - 1):
            cs_cw = (my - s) % N_DEV
            cr_cw = (my - s - 1) % N_DEV
            cs_ccw = (my + s) % N_DEV
            cr_ccw = (my + s + 1) % N_DEV
            if s == 0:
                src_cw = p_ref.at[pl.ds(cs_cw * CHUNK, HC), :]
                src_ccw = p_ref.at[pl.ds(cs_ccw * CHUNK + HC, HC), :]
            else:
                src_cw = cwb.at[s - 1]
                src_ccw = ccwb.at[s - 1]
            r_cw = pltpu.make_async_remote_copy(
                src_ref=src_cw,
                dst_ref=cwb.at[s],
                send_sem=rs_send.at[s, 0],
                recv_sem=rs_recv.at[s, 0],
                device_id=(right,),
                device_id_type=pl.DeviceIdType.MESH,
            )
            r_ccw = pltpu.make_async_remote_copy(
                src_ref=src_ccw,
                dst_ref=ccwb.at[s],
                send_sem=rs_send.at[s, 1],
                recv_sem=rs_recv.at[s, 1],
                device_id=(left,),
                device_id_type=pl.DeviceIdType.MESH,
            )
            r_cw.start()
            r_ccw.start()
            r_cw.wait_recv()
            r_ccw.wait_recv()
            descs += [r_cw, r_ccw]
            cwb[s, :, :] = cwb[s, :, :] + p_ref[pl.ds(cr_cw * CHUNK, HC), :]
            ccwb[s, :, :] = (
                ccwb[s, :, :] + p_ref[pl.ds(cr_ccw * CHUNK + HC, HC), :]
            )

        own_cw = (my + 1) % N_DEV
        own_ccw = (my - 1) % N_DEV
        out_ref[pl.ds(own_cw * CHUNK, HC), :] = cwb[N_DEV - 2, :, :]
        out_ref[pl.ds(own_ccw * CHUNK + HC, HC), :] = ccwb[N_DEV - 2, :, :]

        for h in range(N_DEV - 1):
            ca_cw = (my + 1 - h) % N_DEV
            ca_ccw = (my - 1 + h) % N_DEV
            r_cw = pltpu.make_async_remote_copy(
                src_ref=out_ref.at[pl.ds(ca_cw * CHUNK, HC), :],
                dst_ref=out_ref.at[pl.ds(ca_cw * CHUNK, HC), :],
                send_sem=ag_send.at[h, 0],
                recv_sem=ag_recv.at[h, 0],
                device_id=(right,),
                device_id_type=pl.DeviceIdType.MESH,
            )
            r_ccw = pltpu.make_async_remote_copy(
                src_ref=out_ref.at[pl.ds(ca_ccw * CHUNK + HC, HC), :],
                dst_ref=out_ref.at[pl.ds(ca_ccw * CHUNK + HC, HC), :],
                send_sem=ag_send.at[h, 1],
                recv_sem=ag_recv.at[h, 1],
                device_id=(left,),
                device_id_type=pl.DeviceIdType.MESH,
            )
            r_cw.start()
            r_ccw.start()
            r_cw.wait_recv()
            r_ccw.wait_recv()
            descs += [r_cw, r_ccw]

        for rdma in descs:
            rdma.wait_send()

    return pl.pallas_call(
        body,
        out_shape=jax.ShapeDtypeStruct((N_TOK, D_MODEL), jnp.float32),
        in_specs=[pl.BlockSpec(memory_space=pltpu.VMEM)],
        out_specs=pl.BlockSpec(memory_space=pltpu.VMEM),
        scratch_shapes=[
            pltpu.VMEM((N_DEV - 1, HC, D_MODEL), jnp.float32),
            pltpu.VMEM((N_DEV - 1, HC, D_MODEL), jnp.float32),
            pltpu.SemaphoreType.DMA((N_DEV - 1, 2)),
            pltpu.SemaphoreType.DMA((N_DEV - 1, 2)),
            pltpu.SemaphoreType.DMA((N_DEV - 1, 2)),
            pltpu.SemaphoreType.DMA((N_DEV - 1, 2)),
        ],
        compiler_params=pltpu.CompilerParams(collective_id=0),
    )(partial)


def kernel(x, router_W, route_idx, expert_W):
    my = lax.axis_index("i")

    scores = x @ router_W
    m = scores.max(axis=-1, keepdims=True)
    p = jnp.exp(scores - m)
    p = p / p.sum(axis=-1, keepdims=True)
    g = jnp.take_along_axis(p, route_idx, axis=1)
    w = g / g.sum(axis=-1, keepdims=True)

    local_e = my * E_LOCAL + jnp.arange(E_LOCAL, dtype=jnp.int32)
    hit0 = route_idx[:, 0][None, :] == local_e[:, None]
    hit1 = route_idx[:, 1][None, :] == local_e[:, None]
    c = (w[:, 0][None, :] * hit0 + w[:, 1][None, :] * hit1).astype(jnp.float32)

    partial = _moe_partial(x, expert_W, c)
    return _ring_all_reduce(partial)


# device time: 143708 ns/iter; 1.7755x vs baseline; 1.2720x over previous
import jax
import jax.numpy as jnp
from jax import lax
from jax.experimental import pallas as pl
from jax.experimental.pallas import tpu as pltpu

N_DEV = 8
E_LOCAL = 8
N_TOK = 2048
D_MODEL = 1024
CHUNK = N_TOK // N_DEV


def _moe_partial(x, expert_W, c):

    def body(x_ref, w_ref, c_ref, out_ref):
        e = pl.program_id(0)
        prod = jnp.dot(
            x_ref[:, :], w_ref[0, :, :], preferred_element_type=jnp.float32
        )
        contrib = prod * c_ref[e][:, None]

        @pl.when(e == 0)
        def _():
            out_ref[:, :] = contrib

        @pl.when(e != 0)
        def _():
            out_ref[:, :] = out_ref[:, :] + contrib

    return pl.pallas_call(
        body,
        grid=(E_LOCAL,),
        in_specs=[
            pl.BlockSpec((N_TOK, D_MODEL), lambda e: (0, 0)),
            pl.BlockSpec((1, D_MODEL, D_MODEL), lambda e: (e, 0, 0)),
            pl.BlockSpec((E_LOCAL, N_TOK), lambda e: (0, 0)),
        ],
        out_specs=pl.BlockSpec((N_TOK, D_MODEL), lambda e: (0, 0)),
        out_shape=jax.ShapeDtypeStruct((N_TOK, D_MODEL), jnp.float32),
    )(x, expert_W, c)


def _ring_all_reduce(partial):
    HC = CHUNK // 2

    def body(
        p_ref,
        out_ref,
        cwb,
        ccwb,
        snd_cw,
        snd_ccw,
        agb_cw,
        agb_ccw,
        rs_send,
        rs_recv,
        ag_send,
        ag_recv,
    ):
        my = lax.axis_index("i")
        left = (my - 1) % N_DEV
        right = (my + 1) % N_DEV

        barrier_sem = pltpu.get_barrier_semaphore()
        for nbr in (left, right):
            pl.semaphore_signal(
                barrier_sem,
                inc=1,
                device_id=(nbr,),
                device_id_type=pl.DeviceIdType.MESH,
            )
        pl.semaphore_wait(barrier_sem, 2)

        descs = []

        snd_cw[0, :, :] = p_ref[pl.ds(my * CHUNK, HC), :].astype(jnp.bfloat16)
        snd_ccw[0, :, :] = p_ref[pl.ds(my * CHUNK + HC, HC), :].astype(
            jnp.bfloat16
        )
        for s in range(N_DEV - 1):
            cr_cw = (my - s - 1) % N_DEV
            cr_ccw = (my + s + 1) % N_DEV
            r_cw = pltpu.make_async_remote_copy(
                src_ref=snd_cw.at[s],
                dst_ref=cwb.at[s],
                send_sem=rs_send.at[s, 0],
                recv_sem=rs_recv.at[s, 0],
                device_id=(right,),
                device_id_type=pl.DeviceIdType.MESH,
            )
            r_ccw = pltpu.make_async_remote_copy(
                src_ref=snd_ccw.at[s],
                dst_ref=ccwb.at[s],
                send_sem=rs_send.at[s, 1],
                recv_sem=rs_recv.at[s, 1],
                device_id=(left,),
                device_id_type=pl.DeviceIdType.MESH,
            )
            r_cw.start()
            r_ccw.start()
            r_cw.wait_recv()
            r_ccw.wait_recv()
            descs += [r_cw, r_ccw]
            acc_cw = (
                cwb[s, :, :].astype(jnp.float32)
                + p_ref[pl.ds(cr_cw * CHUNK, HC), :]
            )
            acc_ccw = (
                ccwb[s, :, :].astype(jnp.float32)
                + p_ref[pl.ds(cr_ccw * CHUNK + HC, HC), :]
            )
            if s < N_DEV - 2:
                snd_cw[s + 1, :, :] = acc_cw.astype(jnp.bfloat16)
                snd_ccw[s + 1, :, :] = acc_ccw.astype(jnp.bfloat16)
            else:
                out_ref[pl.ds(cr_cw * CHUNK, HC), :] = acc_cw
                out_ref[pl.ds(cr_ccw * CHUNK + HC, HC), :] = acc_ccw
                agb_cw[0, :, :] = acc_cw.astype(jnp.bfloat16)
                agb_ccw[0, :, :] = acc_ccw.astype(jnp.bfloat16)

        for h in range(N_DEV - 1):
            cg_cw = (my - h) % N_DEV
            cg_ccw = (my + h) % N_DEV
            r_cw = pltpu.make_async_remote_copy(
                src_ref=agb_cw.at[h],
                dst_ref=agb_cw.at[h + 1],
                send_sem=ag_send.at[h, 0],
                recv_sem=ag_recv.at[h, 0],
                device_id=(right,),
                device_id_type=pl.DeviceIdType.MESH,
            )
            r_ccw = pltpu.make_async_remote_copy(
                src_ref=agb_ccw.at[h],
                dst_ref=agb_ccw.at[h + 1],
                send_sem=ag_send.at[h, 1],
                recv_sem=ag_recv.at[h, 1],
                device_id=(left,),
                device_id_type=pl.DeviceIdType.MESH,
            )
            r_cw.start()
            r_ccw.start()
            r_cw.wait_recv()
            r_ccw.wait_recv()
            descs += [r_cw, r_ccw]
            out_ref[pl.ds(cg_cw * CHUNK, HC), :] = agb_cw[
                h + 1, :, :
            ].astype(jnp.float32)
            out_ref[pl.ds(cg_ccw * CHUNK + HC, HC), :] = agb_ccw[
                h + 1, :, :
            ].astype(jnp.float32)

        for rdma in descs:
            rdma.wait_send()

    return pl.pallas_call(
        body,
        out_shape=jax.ShapeDtypeStruct((N_TOK, D_MODEL), jnp.float32),
        in_specs=[pl.BlockSpec(memory_space=pltpu.VMEM)],
        out_specs=pl.BlockSpec(memory_space=pltpu.VMEM),
        scratch_shapes=[
            pltpu.VMEM((N_DEV - 1, HC, D_MODEL), jnp.bfloat16),
            pltpu.VMEM((N_DEV - 1, HC, D_MODEL), jnp.bfloat16),
            pltpu.VMEM((N_DEV - 1, HC, D_MODEL), jnp.bfloat16),
            pltpu.VMEM((N_DEV - 1, HC, D_MODEL), jnp.bfloat16),
            pltpu.VMEM((N_DEV, HC, D_MODEL), jnp.bfloat16),
            pltpu.VMEM((N_DEV, HC, D_MODEL), jnp.bfloat16),
            pltpu.SemaphoreType.DMA((N_DEV - 1, 2)),
            pltpu.SemaphoreType.DMA((N_DEV - 1, 2)),
            pltpu.SemaphoreType.DMA((N_DEV - 1, 2)),
            pltpu.SemaphoreType.DMA((N_DEV - 1, 2)),
        ],
        compiler_params=pltpu.CompilerParams(collective_id=0),
    )(partial)


def kernel(x, router_W, route_idx, expert_W):
    my = lax.axis_index("i")

    scores = x @ router_W
    m = scores.max(axis=-1, keepdims=True)
    p = jnp.exp(scores - m)
    p = p / p.sum(axis=-1, keepdims=True)
    g = jnp.take_along_axis(p, route_idx, axis=1)
    w = g / g.sum(axis=-1, keepdims=True)

    local_e = my * E_LOCAL + jnp.arange(E_LOCAL, dtype=jnp.int32)
    hit0 = route_idx[:, 0][None, :] == local_e[:, None]
    hit1 = route_idx[:, 1][None, :] == local_e[:, None]
    c = (w[:, 0][None, :] * hit0 + w[:, 1][None, :] * hit1).astype(jnp.float32)

    partial = _moe_partial(x, expert_W, c)
    return _ring_all_reduce(partial)


# device time: 132765 ns/iter; 1.9219x vs baseline; 1.0824x over previous
import jax
import jax.numpy as jnp
from jax import lax
from jax.experimental import pallas as pl
from jax.experimental.pallas import tpu as pltpu

N_DEV = 8
E_LOCAL = 8
N_EXP = 64
N_TOK = 2048
D_MODEL = 1024
CHUNK = N_TOK // N_DEV


def _moe_partial(x, router_W, route_idx, expert_W):

    def body(x_ref, rw_ref, idx_ref, w_ref, out_ref, gate_scr):
        e = pl.program_id(0)
        my = lax.axis_index("i")

        @pl.when(e == 0)
        def _():
            scores = jnp.dot(
                x_ref[:, :], rw_ref[:, :], preferred_element_type=jnp.float32
            )
            m = jnp.max(scores, axis=1, keepdims=True)
            p = jnp.exp(scores - m)
            iota = lax.broadcasted_iota(jnp.int32, (N_TOK, N_EXP), 1)
            hit0 = iota == idx_ref[:, 0:1]
            hit1 = iota == idx_ref[:, 1:2]
            g0 = jnp.sum(jnp.where(hit0, p, 0.0), axis=1, keepdims=True)
            g1 = jnp.sum(jnp.where(hit1, p, 0.0), axis=1, keepdims=True)
            gs = g0 + g1
            gate_scr[:, 0:1] = g0 / gs
            gate_scr[:, 1:2] = g1 / gs

        le = my * E_LOCAL + e
        c = jnp.where(idx_ref[:, 0:1] == le, gate_scr[:, 0:1], 0.0) + jnp.where(
            idx_ref[:, 1:2] == le, gate_scr[:, 1:2], 0.0
        )
        prod = jnp.dot(
            x_ref[:, :], w_ref[0, :, :], preferred_element_type=jnp.float32
        )
        contrib = prod * c

        @pl.when(e == 0)
        def _():
            out_ref[:, :] = contrib

        @pl.when(e != 0)
        def _():
            out_ref[:, :] = out_ref[:, :] + contrib

    return pl.pallas_call(
        body,
        grid=(E_LOCAL,),
        in_specs=[
            pl.BlockSpec((N_TOK, D_MODEL), lambda e: (0, 0)),
            pl.BlockSpec((D_MODEL, N_EXP), lambda e: (0, 0)),
            pl.BlockSpec((N_TOK, 2), lambda e: (0, 0)),
            pl.BlockSpec((1, D_MODEL, D_MODEL), lambda e: (e, 0, 0)),
        ],
        out_specs=pl.BlockSpec((N_TOK, D_MODEL), lambda e: (0, 0)),
        out_shape=jax.ShapeDtypeStruct((N_TOK, D_MODEL), jnp.float32),
        scratch_shapes=[pltpu.VMEM((N_TOK, 2), jnp.float32)],
    )(x, router_W, route_idx, expert_W)


def _ring_all_reduce(partial):
    HC = CHUNK // 2

    def body(
        p_ref,
        out_ref,
        cwb,
        ccwb,
        snd_cw,
        snd_ccw,
        agb_cw,
        agb_ccw,
        rs_send,
        rs_recv,
        ag_send,
        ag_recv,
    ):
        my = lax.axis_index("i")
        left = (my - 1) % N_DEV
        right = (my + 1) % N_DEV

        barrier_sem = pltpu.get_barrier_semaphore()
        for nbr in (left, right):
            pl.semaphore_signal(
                barrier_sem,
                inc=1,
                device_id=(nbr,),
                device_id_type=pl.DeviceIdType.MESH,
            )
        pl.semaphore_wait(barrier_sem, 2)

        descs = []

        snd_cw[0, :, :] = p_ref[pl.ds(my * CHUNK, HC), :].astype(jnp.bfloat16)
        snd_ccw[0, :, :] = p_ref[pl.ds(my * CHUNK + HC, HC), :].astype(
            jnp.bfloat16
        )
        for s in range(N_DEV - 1):
            cr_cw = (my - s - 1) % N_DEV
            cr_ccw = (my + s + 1) % N_DEV
            r_cw = pltpu.make_async_remote_copy(
                src_ref=snd_cw.at[s],
                dst_ref=cwb.at[s],
                send_sem=rs_send.at[s, 0],
                recv_sem=rs_recv.at[s, 0],
                device_id=(right,),
                device_id_type=pl.DeviceIdType.MESH,
            )
            r_ccw = pltpu.make_async_remote_copy(
                src_ref=snd_ccw.at[s],
                dst_ref=ccwb.at[s],
                send_sem=rs_send.at[s, 1],
                recv_sem=rs_recv.at[s, 1],
                device_id=(left,),
                device_id_type=pl.DeviceIdType.MESH,
            )
            r_cw.start()
            r_ccw.start()
            r_cw.wait_recv()
            r_ccw.wait_recv()
            descs += [r_cw, r_ccw]
            acc_cw = (
                cwb[s, :, :].astype(jnp.float32)
                + p_ref[pl.ds(cr_cw * CHUNK, HC), :]
            )
            acc_ccw = (
                ccwb[s, :, :].astype(jnp.float32)
                + p_ref[pl.ds(cr_ccw * CHUNK + HC, HC), :]
            )
            if s < N_DEV - 2:
                snd_cw[s + 1, :, :] = acc_cw.astype(jnp.bfloat16)
                snd_ccw[s + 1, :, :] = acc_ccw.astype(jnp.bfloat16)
            else:
                out_ref[pl.ds(cr_cw * CHUNK, HC), :] = acc_cw
                out_ref[pl.ds(cr_ccw * CHUNK + HC, HC), :] = acc_ccw
                agb_cw[0, :, :] = acc_cw.astype(jnp.bfloat16)
                agb_ccw[0, :, :] = acc_ccw.astype(jnp.bfloat16)

        for h in range(N_DEV - 1):
            cg_cw = (my - h) % N_DEV
            cg_ccw = (my + h) % N_DEV
            r_cw = pltpu.make_async_remote_copy(
                src_ref=agb_cw.at[h],
                dst_ref=agb_cw.at[h + 1],
                send_sem=ag_send.at[h, 0],
                recv_sem=ag_recv.at[h, 0],
                device_id=(right,),
                device_id_type=pl.DeviceIdType.MESH,
            )
            r_ccw = pltpu.make_async_remote_copy(
                src_ref=agb_ccw.at[h],
                dst_ref=agb_ccw.at[h + 1],
                send_sem=ag_send.at[h, 1],
                recv_sem=ag_recv.at[h, 1],
                device_id=(left,),
                device_id_type=pl.DeviceIdType.MESH,
            )
            r_cw.start()
            r_ccw.start()
            r_cw.wait_recv()
            r_ccw.wait_recv()
            descs += [r_cw, r_ccw]
            out_ref[pl.ds(cg_cw * CHUNK, HC), :] = agb_cw[
                h + 1, :, :
            ].astype(jnp.float32)
            out_ref[pl.ds(cg_ccw * CHUNK + HC, HC), :] = agb_ccw[
                h + 1, :, :
            ].astype(jnp.float32)

        for rdma in descs:
            rdma.wait_send()

    return pl.pallas_call(
        body,
        out_shape=jax.ShapeDtypeStruct((N_TOK, D_MODEL), jnp.float32),
        in_specs=[pl.BlockSpec(memory_space=pltpu.VMEM)],
        out_specs=pl.BlockSpec(memory_space=pltpu.VMEM),
        scratch_shapes=[
            pltpu.VMEM((N_DEV - 1, HC, D_MODEL), jnp.bfloat16),
            pltpu.VMEM((N_DEV - 1, HC, D_MODEL), jnp.bfloat16),
            pltpu.VMEM((N_DEV - 1, HC, D_MODEL), jnp.bfloat16),
            pltpu.VMEM((N_DEV - 1, HC, D_MODEL), jnp.bfloat16),
            pltpu.VMEM((N_DEV, HC, D_MODEL), jnp.bfloat16),
            pltpu.VMEM((N_DEV, HC, D_MODEL), jnp.bfloat16),
            pltpu.SemaphoreType.DMA((N_DEV - 1, 2)),
            pltpu.SemaphoreType.DMA((N_DEV - 1, 2)),
            pltpu.SemaphoreType.DMA((N_DEV - 1, 2)),
            pltpu.SemaphoreType.DMA((N_DEV - 1, 2)),
        ],
        compiler_params=pltpu.CompilerParams(collective_id=0),
    )(partial)


def kernel(x, router_W, route_idx, expert_W):
    partial = _moe_partial(x, router_W, route_idx, expert_W)
    return _ring_all_reduce(partial)


# device time: 124183 ns/iter; 2.0547x vs baseline; 1.0691x over previous
import jax
import jax.numpy as jnp
from jax import lax
from jax.experimental import pallas as pl
from jax.experimental.pallas import tpu as pltpu

N_DEV = 8
E_LOCAL = 8
N_EXP = 64
N_TOK = 2048
D_MODEL = 1024
CHUNK = N_TOK // N_DEV


def kernel(x, router_W, route_idx, expert_W):
    def body(
        x_ref,
        rw_ref,
        idx_ref,
        w_ref,
        out_ref,
        gate,
        snd,
        rsb,
        agb,
        rs_send,
        rs_recv,
        ag_send,
        ag_recv,
    ):
        my = lax.axis_index("i")

        barrier_sem = pltpu.get_barrier_semaphore()
        for o in range(1, N_DEV):
            pl.semaphore_signal(
                barrier_sem,
                inc=1,
                device_id=((my + o) % N_DEV,),
                device_id_type=pl.DeviceIdType.MESH,
            )
        pl.semaphore_wait(barrier_sem, N_DEV - 1)

        scores = jnp.dot(
            x_ref[:, :], rw_ref[:, :], preferred_element_type=jnp.float32
        )
        m = jnp.max(scores, axis=1, keepdims=True)
        p = jnp.exp(scores - m)
        iota = lax.broadcasted_iota(jnp.int32, (N_TOK, N_EXP), 1)
        g0 = jnp.sum(jnp.where(iota == idx_ref[:, 0:1], p, 0.0), axis=1, keepdims=True)
        g1 = jnp.sum(jnp.where(iota == idx_ref[:, 1:2], p, 0.0), axis=1, keepdims=True)
        gs = g0 + g1
        gate[:, 0:1] = g0 / gs
        gate[:, 1:2] = g1 / gs

        def compute_chunk(c):
            rows = pl.ds(c * CHUNK, CHUNK)
            xc = x_ref[rows, :]
            idx0 = idx_ref[rows, 0:1]
            idx1 = idx_ref[rows, 1:2]
            w0 = gate[rows, 0:1]
            w1 = gate[rows, 1:2]
            xc_bf = xc.astype(jnp.bfloat16)
            acc = jnp.zeros((CHUNK, D_MODEL), jnp.float32)
            for e in range(E_LOCAL):
                le = my * E_LOCAL + e
                ce = jnp.where(idx0 == le, w0, 0.0) + jnp.where(
                    idx1 == le, w1, 0.0
                )
                acc = acc + ce * jnp.dot(
                    xc_bf, w_ref[e, :, :], preferred_element_type=jnp.float32
                )
            return acc

        descs = []

        for o in range(1, N_DEV):
            dst = (my + o) % N_DEV
            snd[dst, :, :] = compute_chunk(dst).astype(jnp.bfloat16)
            r = pltpu.make_async_remote_copy(
                src_ref=snd.at[dst],
                dst_ref=rsb.at[my],
                send_sem=rs_send.at[dst],
                recv_sem=rs_recv.at[my],
                device_id=(dst,),
                device_id_type=pl.DeviceIdType.MESH,
            )
            r.start()
            descs.append(r)

        acc = compute_chunk(my)
        for o in range(1, N_DEV):
            src_dev = (my + o) % N_DEV
            rwait = pltpu.make_async_remote_copy(
                src_ref=snd.at[src_dev],
                dst_ref=rsb.at[src_dev],
                send_sem=rs_send.at[src_dev],
                recv_sem=rs_recv.at[src_dev],
                device_id=(src_dev,),
                device_id_type=pl.DeviceIdType.MESH,
            )
            rwait.wait_recv()
            acc = acc + rsb[src_dev, :, :].astype(jnp.float32)
        out_ref[pl.ds(my * CHUNK, CHUNK), :] = acc
        snd[my, :, :] = acc.astype(jnp.bfloat16)

        for o in range(1, N_DEV):
            dst = (my + o) % N_DEV
            r = pltpu.make_async_remote_copy(
                src_ref=snd.at[my],
                dst_ref=agb.at[my],
                send_sem=ag_send.at[dst],
                recv_sem=ag_recv.at[my],
                device_id=(dst,),
                device_id_type=pl.DeviceIdType.MESH,
            )
            r.start()
            descs.append(r)
        for o in range(1, N_DEV):
            src_dev = (my + o) % N_DEV
            rwait = pltpu.make_async_remote_copy(
                src_ref=snd.at[src_dev],
                dst_ref=agb.at[src_dev],
                send_sem=ag_send.at[src_dev],
                recv_sem=ag_recv.at[src_dev],
                device_id=(src_dev,),
                device_id_type=pl.DeviceIdType.MESH,
            )
            rwait.wait_recv()
            out_ref[pl.ds(src_dev * CHUNK, CHUNK), :] = agb[
                src_dev, :, :
            ].astype(jnp.float32)

        for r in descs:
            r.wait_send()

    return pl.pallas_call(
        body,
        out_shape=jax.ShapeDtypeStruct((N_TOK, D_MODEL), jnp.float32),
        in_specs=[
            pl.BlockSpec(memory_space=pltpu.VMEM),
            pl.BlockSpec(memory_space=pltpu.VMEM),
            pl.BlockSpec(memory_space=pltpu.VMEM),
            pl.BlockSpec(memory_space=pltpu.VMEM),
        ],
        out_specs=pl.BlockSpec(memory_space=pltpu.VMEM),
        scratch_shapes=[
            pltpu.VMEM((N_TOK, 2), jnp.float32),
            pltpu.VMEM((N_DEV, CHUNK, D_MODEL), jnp.bfloat16),
            pltpu.VMEM((N_DEV, CHUNK, D_MODEL), jnp.bfloat16),
            pltpu.VMEM((N_DEV, CHUNK, D_MODEL), jnp.bfloat16),
            pltpu.SemaphoreType.DMA((N_DEV,)),
            pltpu.SemaphoreType.DMA((N_DEV,)),
            pltpu.SemaphoreType.DMA((N_DEV,)),
            pltpu.SemaphoreType.DMA((N_DEV,)),
        ],
        compiler_params=pltpu.CompilerParams(
            collective_id=0, vmem_limit_bytes=100 * 1024 * 1024
        ),
    )(x, router_W, route_idx, expert_W.astype(jnp.bfloat16))
